# baseline (speedup 1.0000x reference)
import functools

import jax
import jax.numpy as jnp
from jax import lax
from jax.experimental import pallas as pl
from jax.experimental.pallas import tpu as pltpu
from jax.experimental.pallas import tpu_sc as plsc

N_NODES = 100000
EMB_DIM = 128
NUM_EMB = 119
WINDOW = 200
NUM_BLOCKS = N_NODES // WINDOW
NUM_WORKERS = 32
BLOCKS_PER_WORKER = -(-NUM_BLOCKS // NUM_WORKERS)


def kernel(x, table):
    idx = x.reshape(N_NODES).astype(jnp.int32)
    mesh = plsc.VectorSubcoreMesh(core_axis_name="c", subcore_axis_name="s")

    @functools.partial(
        pl.kernel,
        out_type=jax.ShapeDtypeStruct((N_NODES, EMB_DIM), jnp.float32),
        mesh=mesh,
        scratch_types=[
            pltpu.VMEM_SHARED((NUM_EMB, EMB_DIM), jnp.float32),
            pltpu.VMEM((WINDOW,), jnp.int32),
            pltpu.VMEM((WINDOW,), jnp.int32),
            pltpu.VMEM((WINDOW, EMB_DIM), jnp.float32),
            pltpu.VMEM((WINDOW, EMB_DIM), jnp.float32),
            pltpu.SemaphoreType.DMA((2,)),
            pltpu.SemaphoreType.DMA((2,)),
            pltpu.SemaphoreType.DMA((2,)),
        ],
    )
    def gather_kernel(table_hbm, idx_hbm, out_hbm, table_sp, idx_v0, idx_v1,
                      rows_v0, rows_v1, isem, gsem, wsem):
        wid = lax.axis_index("s") * 2 + lax.axis_index("c")
        nb = BLOCKS_PER_WORKER
        idx_bufs = (idx_v0, idx_v1)
        row_bufs = (rows_v0, rows_v1)

        @pl.when(lax.axis_index("s") == 0)
        def _():
            pltpu.sync_copy(table_hbm, table_sp)

        plsc.subcore_barrier()

        def base(j):
            return (wid + j * NUM_WORKERS) * WINDOW

        def idx_copy(j):
            k = j % 2
            return pltpu.make_async_copy(
                idx_hbm.at[pl.ds(base(j), WINDOW)], idx_bufs[k], isem.at[k])

        def gather_copy(j):
            k = j % 2
            return pltpu.make_async_copy(
                table_sp.at[idx_bufs[k]], row_bufs[k], gsem.at[k])

        def write_copy(j):
            k = j % 2
            return pltpu.make_async_copy(
                row_bufs[k], out_hbm.at[pl.ds(base(j), WINDOW)], wsem.at[k])

        def guarded(j, fn):
            if j == nb - 1:
                @pl.when(base(j) < N_NODES)
                def _():
                    fn()
            else:
                fn()

        guarded(0, lambda: idx_copy(0).start())
        for j in range(nb):
            if j + 1 < nb:
                guarded(j + 1, lambda: idx_copy(j + 1).start())
            if j >= 2:
                guarded(j - 2, lambda: write_copy(j - 2).wait())
            guarded(j, lambda: idx_copy(j).wait())
            guarded(j, lambda: gather_copy(j).start())
            guarded(j, lambda: gather_copy(j).wait())
            guarded(j, lambda: write_copy(j).start())
        guarded(nb - 2, lambda: write_copy(nb - 2).wait())
        guarded(nb - 1, lambda: write_copy(nb - 1).wait())

    return gather_kernel(table, idx)


# speedup vs baseline: 1.0119x; 1.0119x over previous
import functools

import jax
import jax.numpy as jnp
from jax import lax
from jax.experimental import pallas as pl
from jax.experimental.pallas import tpu as pltpu
from jax.experimental.pallas import tpu_sc as plsc

N_NODES = 100000
EMB_DIM = 128
NUM_EMB = 119
WINDOW = 400
NUM_BLOCKS = N_NODES // WINDOW
NUM_WORKERS = 32
BLOCKS_PER_WORKER = -(-NUM_BLOCKS // NUM_WORKERS)


def kernel(x, table):
    idx = x.reshape(N_NODES).astype(jnp.int32)
    mesh = plsc.VectorSubcoreMesh(core_axis_name="c", subcore_axis_name="s")

    @functools.partial(
        pl.kernel,
        out_type=jax.ShapeDtypeStruct((N_NODES, EMB_DIM), jnp.float32),
        mesh=mesh,
        scratch_types=[
            pltpu.VMEM_SHARED((NUM_EMB, EMB_DIM), jnp.float32),
            pltpu.VMEM((WINDOW,), jnp.int32),
            pltpu.VMEM((WINDOW,), jnp.int32),
            pltpu.VMEM((WINDOW, EMB_DIM), jnp.float32),
            pltpu.VMEM((WINDOW, EMB_DIM), jnp.float32),
            pltpu.SemaphoreType.DMA((2,)),
            pltpu.SemaphoreType.DMA((2,)),
            pltpu.SemaphoreType.DMA((2,)),
        ],
    )
    def gather_kernel(table_hbm, idx_hbm, out_hbm, table_sp, idx_v0, idx_v1,
                      rows_v0, rows_v1, isem, gsem, wsem):
        wid = lax.axis_index("s") * 2 + lax.axis_index("c")
        nb = BLOCKS_PER_WORKER
        idx_bufs = (idx_v0, idx_v1)
        row_bufs = (rows_v0, rows_v1)

        @pl.when(lax.axis_index("s") == 0)
        def _():
            pltpu.sync_copy(table_hbm, table_sp)

        plsc.subcore_barrier()

        def base(j):
            return (wid + j * NUM_WORKERS) * WINDOW

        def idx_copy(j):
            k = j % 2
            return pltpu.make_async_copy(
                idx_hbm.at[pl.ds(base(j), WINDOW)], idx_bufs[k], isem.at[k])

        def gather_copy(j):
            k = j % 2
            return pltpu.make_async_copy(
                table_sp.at[idx_bufs[k]], row_bufs[k], gsem.at[k])

        def write_copy(j):
            k = j % 2
            return pltpu.make_async_copy(
                row_bufs[k], out_hbm.at[pl.ds(base(j), WINDOW)], wsem.at[k])

        def guarded(j, fn):
            if j == nb - 1:
                @pl.when(base(j) < N_NODES)
                def _():
                    fn()
            else:
                fn()

        guarded(0, lambda: idx_copy(0).start())
        for j in range(nb):
            if j + 1 < nb:
                guarded(j + 1, lambda: idx_copy(j + 1).start())
            if j >= 2:
                guarded(j - 2, lambda: write_copy(j - 2).wait())
            guarded(j, lambda: idx_copy(j).wait())
            guarded(j, lambda: gather_copy(j).start())
            guarded(j, lambda: gather_copy(j).wait())
            guarded(j, lambda: write_copy(j).start())
        guarded(nb - 2, lambda: write_copy(nb - 2).wait())
        guarded(nb - 1, lambda: write_copy(nb - 1).wait())

    return gather_kernel(table, idx)


# speedup vs baseline: 1.0173x; 1.0053x over previous
import functools

import jax
import jax.numpy as jnp
from jax import lax
from jax.experimental import pallas as pl
from jax.experimental.pallas import tpu as pltpu
from jax.experimental.pallas import tpu_sc as plsc

N_NODES = 100000
EMB_DIM = 128
NUM_EMB = 119
NBUF = 4
WINDOW = 200
NUM_BLOCKS = N_NODES // WINDOW
NUM_WORKERS = 32
BLOCKS_PER_WORKER = -(-NUM_BLOCKS // NUM_WORKERS)


def kernel(x, table):
    idx = x.reshape(N_NODES).astype(jnp.int32)
    mesh = plsc.VectorSubcoreMesh(core_axis_name="c", subcore_axis_name="s")

    @functools.partial(
        pl.kernel,
        out_type=jax.ShapeDtypeStruct((N_NODES, EMB_DIM), jnp.float32),
        mesh=mesh,
        scratch_types=(
            [pltpu.VMEM_SHARED((NUM_EMB, EMB_DIM), jnp.float32)]
            + [pltpu.VMEM((WINDOW,), jnp.int32) for _ in range(NBUF)]
            + [pltpu.VMEM((WINDOW, EMB_DIM), jnp.float32) for _ in range(NBUF)]
            + [
                pltpu.SemaphoreType.DMA((NBUF,)),
                pltpu.SemaphoreType.DMA((NBUF,)),
                pltpu.SemaphoreType.DMA((NBUF,)),
            ]
        ),
    )
    def gather_kernel(table_hbm, idx_hbm, out_hbm, table_sp, *rest):
        idx_bufs = rest[:NBUF]
        row_bufs = rest[NBUF:2 * NBUF]
        isem, gsem, wsem = rest[2 * NBUF:]
        wid = lax.axis_index("s") * 2 + lax.axis_index("c")
        nb = BLOCKS_PER_WORKER

        @pl.when(lax.axis_index("s") == 0)
        def _():
            pltpu.sync_copy(table_hbm, table_sp)

        plsc.subcore_barrier()

        def base(j):
            return (wid + j * NUM_WORKERS) * WINDOW

        def idx_copy(j):
            k = j % NBUF
            return pltpu.make_async_copy(
                idx_hbm.at[pl.ds(base(j), WINDOW)], idx_bufs[k], isem.at[k])

        def gather_copy(j):
            k = j % NBUF
            return pltpu.make_async_copy(
                table_sp.at[idx_bufs[k]], row_bufs[k], gsem.at[k])

        def write_copy(j):
            k = j % NBUF
            return pltpu.make_async_copy(
                row_bufs[k], out_hbm.at[pl.ds(base(j), WINDOW)], wsem.at[k])

        def guarded(j, fn):
            if j == nb - 1:
                @pl.when(base(j) < N_NODES)
                def _():
                    fn()
            else:
                fn()

        guarded(0, lambda: idx_copy(0).start())
        guarded(1, lambda: idx_copy(1).start())
        guarded(0, lambda: idx_copy(0).wait())
        guarded(0, lambda: gather_copy(0).start())
        for j in range(nb):
            if j + 2 < nb:
                guarded(j + 2, lambda: idx_copy(j + 2).start())
            if j + 1 < nb:
                guarded(j + 1, lambda: idx_copy(j + 1).wait())
                if j >= 3:
                    guarded(j - 3, lambda: write_copy(j - 3).wait())
                guarded(j + 1, lambda: gather_copy(j + 1).start())
            guarded(j, lambda: gather_copy(j).wait())
            guarded(j, lambda: write_copy(j).start())
        for j in range(max(0, nb - 4), nb):
            guarded(j, lambda: write_copy(j).wait())

    return gather_kernel(table, idx)
